# baseline (speedup 1.0000x reference)
import jax
import jax.numpy as jnp
from jax.experimental import pallas as pl
from jax.experimental.pallas import tpu as pltpu

_PACK = 16


def _round_up(x, m):
    return ((x + m - 1) // m) * m


def _mlp_packed_body(xp_ref, w1e_ref, b1e_ref, w2e_ref, b2e_ref,
                     w3cm_ref, b3cm_ref, w3bm_ref, b3bm_ref, t3_ref, o_ref):
    xp = xp_ref[...]
    h = jnp.dot(xp, w1e_ref[...], preferred_element_type=jnp.float32)
    h = jnp.maximum(h + b1e_ref[...], 0.0)
    h = jnp.dot(h, w2e_ref[...], preferred_element_type=jnp.float32)
    h = jnp.maximum(h + b2e_ref[...], 0.0)

    lg = jnp.dot(h, w3cm_ref[...], preferred_element_type=jnp.float32)
    lg = lg + b3cm_ref[...]
    l0, l1, l2 = lg[:, 0:16], lg[:, 16:32], lg[:, 32:48]
    m = jnp.maximum(jnp.maximum(l0, l1), l2)
    se = jnp.exp(l0 - m) + jnp.exp(l1 - m) + jnp.exp(l2 - m)
    lse = m + jnp.log(se)

    ob = jnp.dot(h, w3bm_ref[...], preferred_element_type=jnp.float32)
    ob = ob + b3bm_ref[...]
    ob = ob - jnp.dot(lse, t3_ref[...], preferred_element_type=jnp.float32)
    o_ref[...] = ob.astype(o_ref.dtype)


def kernel(x, w1, b1, w2, b2, w3, b3, *, block_rows=2048):
    B, n_in = x.shape
    h1, h2, n_out = w1.shape[0], w2.shape[0], w3.shape[0]

    chunk = _PACK * block_rows
    Bp = _round_up(B, chunk)
    if Bp != B:
        x = jnp.pad(x, ((0, Bp - B), (0, 0)))
    R = Bp // _PACK
    xp = x.reshape(R, _PACK * n_in)

    eye = jnp.eye(_PACK, dtype=jnp.float32)
    w1e = jnp.kron(eye, w1.T)
    w2e = jnp.kron(eye, w2.T)
    w3bm = jnp.kron(eye, w3.T)
    w3cm = jnp.einsum('bd,hc->bhcd', eye, w3.T).reshape(_PACK * h2,
                                                        _PACK * n_out)
    t3 = jnp.kron(eye, jnp.ones((1, n_out), jnp.float32))
    b1e = jnp.tile(b1.reshape(-1), _PACK)[None]
    b2e = jnp.tile(b2.reshape(-1), _PACK)[None]
    b3bm = jnp.tile(b3.reshape(-1), _PACK)[None]
    b3cm = jnp.repeat(b3.reshape(-1), _PACK)[None]

    grid = (R // block_rows,)
    flops = 2 * Bp * (n_in * h1 + h1 * h2 + h2 * n_out)
    transcendentals = Bp * (n_out + 1)
    bytes_accessed = (Bp * n_in * 4 + Bp * n_out * 4
                      + (w1e.size + w2e.size + w3cm.size + w3bm.size) * 4)

    const = lambda i: (0, 0)
    wspec = [pl.BlockSpec(a.shape, const)
             for a in (w1e, b1e, w2e, b2e, w3cm, b3cm, w3bm, b3bm, t3)]
    op = pl.pallas_call(
        _mlp_packed_body,
        out_shape=jax.ShapeDtypeStruct((R, _PACK * n_out), jnp.float32),
        grid=grid,
        in_specs=[pl.BlockSpec((block_rows, _PACK * n_in),
                               lambda i: (i, 0))] + wspec,
        out_specs=pl.BlockSpec((block_rows, _PACK * n_out), lambda i: (i, 0)),
        compiler_params=pltpu.CompilerParams(
            dimension_semantics=("parallel",)),
        cost_estimate=pl.CostEstimate(
            flops=flops,
            transcendentals=transcendentals,
            bytes_accessed=bytes_accessed),
    )(xp, w1e, b1e, w2e, b2e, w3cm, b3cm, w3bm, b3bm, t3)

    return op.reshape(Bp, n_out)[:B]


# speedup vs baseline: 9.0185x; 9.0185x over previous
import jax
import jax.numpy as jnp
from jax.experimental import pallas as pl
from jax.experimental.pallas import tpu as pltpu

_LANE = 128


def _round_up(x, m):
    return ((x + m - 1) // m) * m


def _fused_mlp_logsoftmax(xt_ref, w1_ref, b1_ref, w2_ref, b2_ref,
                          w3_ref, b3_ref, o_ref):
    a = jnp.dot(w1_ref[...], xt_ref[...],
                preferred_element_type=jnp.float32)
    a = jnp.maximum(a + b1_ref[...], 0.0)
    a = jnp.dot(w2_ref[...], a,
                preferred_element_type=jnp.float32)
    a = jnp.maximum(a + b2_ref[...], 0.0)
    lg = jnp.dot(w3_ref[...], a,
                 preferred_element_type=jnp.float32) + b3_ref[...]

    m = jnp.max(lg, axis=0, keepdims=True)
    sh = lg - m
    lse = jnp.log(jnp.sum(jnp.exp(sh), axis=0, keepdims=True))
    o_ref[...] = (sh - lse).astype(o_ref.dtype)


def kernel(x, w1, b1, w2, b2, w3, b3, *, block_batch=8192):
    B, n_in = x.shape
    h1, h2, n_out = w1.shape[0], w2.shape[0], w3.shape[0]

    xt = x.T
    BT = max(_LANE, min(block_batch, _round_up(B, _LANE)))
    Bp = _round_up(B, BT)
    if Bp != B:
        xt = jnp.pad(xt, ((0, 0), (0, Bp - B)))
    grid = (Bp // BT,)

    flops = 2 * Bp * (n_in * h1 + h1 * h2 + h2 * n_out)
    transcendentals = Bp * (n_out + 1)
    bytes_accessed = (Bp * n_in * 4 + Bp * n_out * 4
                      + (w1.size + w2.size + w3.size
                         + b1.size + b2.size + b3.size) * 4)

    const = lambda i: (0, 0)
    out_t = pl.pallas_call(
        _fused_mlp_logsoftmax,
        out_shape=jax.ShapeDtypeStruct((n_out, Bp), jnp.float32),
        grid=grid,
        in_specs=[
            pl.BlockSpec((n_in, BT), lambda i: (0, i)),
            pl.BlockSpec(w1.shape, const), pl.BlockSpec(b1.shape, const),
            pl.BlockSpec(w2.shape, const), pl.BlockSpec(b2.shape, const),
            pl.BlockSpec(w3.shape, const), pl.BlockSpec(b3.shape, const),
        ],
        out_specs=pl.BlockSpec((n_out, BT), lambda i: (0, i)),
        compiler_params=pltpu.CompilerParams(
            dimension_semantics=("parallel",)),
        cost_estimate=pl.CostEstimate(
            flops=flops,
            transcendentals=transcendentals,
            bytes_accessed=bytes_accessed),
    )(xt, w1, b1, w2, b2, w3, b3)

    return out_t[:, :B].T


# speedup vs baseline: 15.9466x; 1.7682x over previous
import jax
import jax.numpy as jnp
from jax.experimental import pallas as pl
from jax.experimental.pallas import tpu as pltpu

_LANE = 128


def _round_up(x, m):
    return ((x + m - 1) // m) * m


def _fused_mlp_logsoftmax(xt_ref, w1_ref, b1_ref, w2_ref, b2_ref,
                          w3_ref, b3_ref, o_ref):
    a = jnp.dot(w1_ref[...], xt_ref[...],
                preferred_element_type=jnp.float32)
    a = jnp.maximum(a + b1_ref[...], 0.0)
    a = jnp.dot(w2_ref[...], a,
                preferred_element_type=jnp.float32)
    a = jnp.maximum(a + b2_ref[...], 0.0)
    lg = jnp.dot(w3_ref[...], a,
                 preferred_element_type=jnp.float32) + b3_ref[...]

    m = jnp.max(lg, axis=0, keepdims=True)
    sh = lg - m
    lse = jnp.log(jnp.sum(jnp.exp(sh), axis=0, keepdims=True))
    o_ref[...] = (sh - lse).astype(o_ref.dtype)


def kernel(x, w1, b1, w2, b2, w3, b3, *, block_batch=32768):
    B, n_in = x.shape
    h1, h2, n_out = w1.shape[0], w2.shape[0], w3.shape[0]

    xt = x.T
    BT = max(_LANE, min(block_batch, _round_up(B, _LANE)))
    Bp = _round_up(B, BT)
    if Bp != B:
        xt = jnp.pad(xt, ((0, 0), (0, Bp - B)))
    grid = (Bp // BT,)

    flops = 2 * Bp * (n_in * h1 + h1 * h2 + h2 * n_out)
    transcendentals = Bp * (n_out + 1)
    bytes_accessed = (Bp * n_in * 4 + Bp * n_out * 4
                      + (w1.size + w2.size + w3.size
                         + b1.size + b2.size + b3.size) * 4)

    const = lambda i: (0, 0)
    out_t = pl.pallas_call(
        _fused_mlp_logsoftmax,
        out_shape=jax.ShapeDtypeStruct((n_out, Bp), jnp.float32),
        grid=grid,
        in_specs=[
            pl.BlockSpec((n_in, BT), lambda i: (0, i)),
            pl.BlockSpec(w1.shape, const), pl.BlockSpec(b1.shape, const),
            pl.BlockSpec(w2.shape, const), pl.BlockSpec(b2.shape, const),
            pl.BlockSpec(w3.shape, const), pl.BlockSpec(b3.shape, const),
        ],
        out_specs=pl.BlockSpec((n_out, BT), lambda i: (0, i)),
        compiler_params=pltpu.CompilerParams(
            dimension_semantics=("parallel",)),
        cost_estimate=pl.CostEstimate(
            flops=flops,
            transcendentals=transcendentals,
            bytes_accessed=bytes_accessed),
    )(xt, w1, b1, w2, b2, w3, b3)

    return out_t[:, :B].T


# speedup vs baseline: 16.6278x; 1.0427x over previous
import jax
import jax.numpy as jnp
from jax.experimental import pallas as pl
from jax.experimental.pallas import tpu as pltpu

_LANE = 128


def _round_up(x, m):
    return ((x + m - 1) // m) * m


def _fused_mlp_logsoftmax(xt_ref, w1_ref, b1_ref, w2_ref, b2_ref,
                          w3_ref, b3_ref, o_ref):
    a = jnp.dot(w1_ref[...], xt_ref[...],
                preferred_element_type=jnp.float32)
    a = jnp.maximum(a + b1_ref[...], 0.0)
    a = jnp.dot(w2_ref[...], a,
                preferred_element_type=jnp.float32)
    a = jnp.maximum(a + b2_ref[...], 0.0)
    lg = jnp.dot(w3_ref[...], a,
                 preferred_element_type=jnp.float32) + b3_ref[...]

    m = jnp.max(lg, axis=0, keepdims=True)
    sh = lg - m
    lse = jnp.log(jnp.sum(jnp.exp(sh), axis=0, keepdims=True))
    o_ref[...] = (sh - lse).astype(o_ref.dtype)


def kernel(x, w1, b1, w2, b2, w3, b3, *, block_batch=65536):
    B, n_in = x.shape
    h1, h2, n_out = w1.shape[0], w2.shape[0], w3.shape[0]

    xt = x.T
    BT = max(_LANE, min(block_batch, _round_up(B, _LANE)))
    Bp = _round_up(B, BT)
    if Bp != B:
        xt = jnp.pad(xt, ((0, 0), (0, Bp - B)))
    grid = (Bp // BT,)

    flops = 2 * Bp * (n_in * h1 + h1 * h2 + h2 * n_out)
    transcendentals = Bp * (n_out + 1)
    bytes_accessed = (Bp * n_in * 4 + Bp * n_out * 4
                      + (w1.size + w2.size + w3.size
                         + b1.size + b2.size + b3.size) * 4)

    const = lambda i: (0, 0)
    out_t = pl.pallas_call(
        _fused_mlp_logsoftmax,
        out_shape=jax.ShapeDtypeStruct((n_out, Bp), jnp.float32),
        grid=grid,
        in_specs=[
            pl.BlockSpec((n_in, BT), lambda i: (0, i)),
            pl.BlockSpec(w1.shape, const), pl.BlockSpec(b1.shape, const),
            pl.BlockSpec(w2.shape, const), pl.BlockSpec(b2.shape, const),
            pl.BlockSpec(w3.shape, const), pl.BlockSpec(b3.shape, const),
        ],
        out_specs=pl.BlockSpec((n_out, BT), lambda i: (0, i)),
        compiler_params=pltpu.CompilerParams(
            dimension_semantics=("parallel",)),
        cost_estimate=pl.CostEstimate(
            flops=flops,
            transcendentals=transcendentals,
            bytes_accessed=bytes_accessed),
    )(xt, w1, b1, w2, b2, w3, b3)

    return out_t[:, :B].T


# speedup vs baseline: 16.7036x; 1.0046x over previous
import jax
import jax.numpy as jnp
from jax.experimental import pallas as pl
from jax.experimental.pallas import tpu as pltpu

_LANE = 128


def _round_up(x, m):
    return ((x + m - 1) // m) * m


def _fused_mlp_logsoftmax(xt_ref, w1_ref, b1_ref, w2_ref, b2_ref,
                          w3_ref, b3_ref, o_ref):
    a = jnp.dot(w1_ref[...], xt_ref[...],
                preferred_element_type=jnp.float32)
    a = jnp.maximum(a + b1_ref[...], 0.0)
    a = jnp.dot(w2_ref[...], a,
                preferred_element_type=jnp.float32)
    a = jnp.maximum(a + b2_ref[...], 0.0)
    lg = jnp.dot(w3_ref[...], a,
                 preferred_element_type=jnp.float32) + b3_ref[...]

    m = jnp.max(lg, axis=0, keepdims=True)
    sh = lg - m
    lse = jnp.log(jnp.sum(jnp.exp(sh), axis=0, keepdims=True))
    o_ref[...] = (sh - lse).astype(o_ref.dtype)


def kernel(x, w1, b1, w2, b2, w3, b3, *, block_batch=131072):
    B, n_in = x.shape
    h1, h2, n_out = w1.shape[0], w2.shape[0], w3.shape[0]

    xt = x.T
    BT = max(_LANE, min(block_batch, _round_up(B, _LANE)))
    Bp = _round_up(B, BT)
    if Bp != B:
        xt = jnp.pad(xt, ((0, 0), (0, Bp - B)))
    grid = (Bp // BT,)

    flops = 2 * Bp * (n_in * h1 + h1 * h2 + h2 * n_out)
    transcendentals = Bp * (n_out + 1)
    bytes_accessed = (Bp * n_in * 4 + Bp * n_out * 4
                      + (w1.size + w2.size + w3.size
                         + b1.size + b2.size + b3.size) * 4)

    const = lambda i: (0, 0)
    out_t = pl.pallas_call(
        _fused_mlp_logsoftmax,
        out_shape=jax.ShapeDtypeStruct((n_out, Bp), jnp.float32),
        grid=grid,
        in_specs=[
            pl.BlockSpec((n_in, BT), lambda i: (0, i)),
            pl.BlockSpec(w1.shape, const), pl.BlockSpec(b1.shape, const),
            pl.BlockSpec(w2.shape, const), pl.BlockSpec(b2.shape, const),
            pl.BlockSpec(w3.shape, const), pl.BlockSpec(b3.shape, const),
        ],
        out_specs=pl.BlockSpec((n_out, BT), lambda i: (0, i)),
        compiler_params=pltpu.CompilerParams(
            dimension_semantics=("parallel",)),
        cost_estimate=pl.CostEstimate(
            flops=flops,
            transcendentals=transcendentals,
            bytes_accessed=bytes_accessed),
    )(xt, w1, b1, w2, b2, w3, b3)

    return out_t[:, :B].T


# speedup vs baseline: 17.4834x; 1.0467x over previous
import jax
import jax.numpy as jnp
from jax.experimental import pallas as pl
from jax.experimental.pallas import tpu as pltpu

_LANE = 128


def _round_up(x, m):
    return ((x + m - 1) // m) * m


def _fused_mlp_logsoftmax(xt_ref, w1_ref, nb1_ref, w2_ref, nb2_ref,
                          w3_ref, b3_ref, ones3_ref, o_ref):
    z = jnp.dot(w1_ref[...], xt_ref[...],
                preferred_element_type=jnp.float32)
    u = jnp.maximum(z, nb1_ref[...])
    z = jnp.dot(w2_ref[...], u,
                preferred_element_type=jnp.float32)
    u = jnp.maximum(z, nb2_ref[...])
    lg = jnp.dot(w3_ref[...], u,
                 preferred_element_type=jnp.float32) + b3_ref[...]

    m = jnp.max(lg, axis=0, keepdims=True)
    sh = lg - m
    s = jnp.dot(ones3_ref[...], jnp.exp(sh),
                preferred_element_type=jnp.float32)
    o_ref[...] = (sh - jnp.log(s)).astype(o_ref.dtype)


def kernel(x, w1, b1, w2, b2, w3, b3, *, block_batch=131072):
    B, n_in = x.shape
    h1, h2, n_out = w1.shape[0], w2.shape[0], w3.shape[0]

    xt = x.T
    BT = max(_LANE, min(block_batch, _round_up(B, _LANE)))
    Bp = _round_up(B, BT)
    if Bp != B:
        xt = jnp.pad(xt, ((0, 0), (0, Bp - B)))
    grid = (Bp // BT,)

    b2f = w2 @ b1 + b2
    b3f = w3 @ b2f + b3
    ones3 = jnp.ones((1, n_out), jnp.float32)

    flops = 2 * Bp * (n_in * h1 + h1 * h2 + h2 * n_out)
    transcendentals = Bp * (n_out + 1)
    bytes_accessed = (Bp * n_in * 4 + Bp * n_out * 4
                      + (w1.size + w2.size + w3.size
                         + b1.size + b2.size + b3.size) * 4)

    const = lambda i: (0, 0)
    out_t = pl.pallas_call(
        _fused_mlp_logsoftmax,
        out_shape=jax.ShapeDtypeStruct((n_out, Bp), jnp.float32),
        grid=grid,
        in_specs=[
            pl.BlockSpec((n_in, BT), lambda i: (0, i)),
            pl.BlockSpec(w1.shape, const), pl.BlockSpec(b1.shape, const),
            pl.BlockSpec(w2.shape, const), pl.BlockSpec(b2f.shape, const),
            pl.BlockSpec(w3.shape, const), pl.BlockSpec(b3f.shape, const),
            pl.BlockSpec(ones3.shape, const),
        ],
        out_specs=pl.BlockSpec((n_out, BT), lambda i: (0, i)),
        compiler_params=pltpu.CompilerParams(
            dimension_semantics=("parallel",)),
        cost_estimate=pl.CostEstimate(
            flops=flops,
            transcendentals=transcendentals,
            bytes_accessed=bytes_accessed),
    )(xt, w1, -b1, w2, -b2f, w3, b3f, ones3)

    return out_t[:, :B].T
